# baseline (speedup 1.0000x reference)
import functools

import numpy as np
import jax
import jax.numpy as jnp
from jax import lax
from jax.experimental import pallas as pl
from jax.experimental.pallas import tpu as pltpu
from jax.experimental.pallas import tpu_sc as plsc

H = 16
DH = 64
CAP = 1024

_FR = np.exp(np.linspace(0.0, -1.0, DH // 2) * np.log(10000.0)).astype(np.float32)
_FREQ_ROW = np.tile(np.concatenate([_FR, _FR]), H)[None, :]
_SIN_SEL = np.tile(
    np.concatenate([np.ones(DH // 2, np.float32), np.zeros(DH // 2, np.float32)]), H
)[None, :]


def _router_body(q_ref, w_ref, o_ref):
    o_ref[...] = jnp.dot(q_ref[...], w_ref[...], preferred_element_type=jnp.float32)


def _router(q2, w_router_t):
    R, D = q2.shape
    blk = 1024
    return pl.pallas_call(
        _router_body,
        grid=(R // blk,),
        in_specs=[
            pl.BlockSpec((blk, D), lambda i: (i, 0)),
            pl.BlockSpec((D, 1), lambda i: (0, 0)),
        ],
        out_specs=pl.BlockSpec((blk, 1), lambda i: (i, 0)),
        out_shape=jax.ShapeDtypeStruct((R, 1), jnp.float32),
    )(q2, w_router_t)


def _proj_body(x_ref, wt_ref, o_ref):
    o_ref[...] = jnp.dot(x_ref[...], wt_ref[...], preferred_element_type=jnp.float32)


def _proj(x, wt):
    R, D = x.shape
    N = wt.shape[1]
    blk = 512
    return pl.pallas_call(
        _proj_body,
        grid=(R // blk,),
        in_specs=[
            pl.BlockSpec((blk, D), lambda i: (i, 0)),
            pl.BlockSpec((D, N), lambda i: (0, 0)),
        ],
        out_specs=pl.BlockSpec((blk, N), lambda i: (i, 0)),
        out_shape=jax.ShapeDtypeStruct((R, N), jnp.float32),
    )(x, wt)


def _proj_rope_body(x_ref, wt_ref, pos_ref, freq_ref, sel_ref, o_ref):
    y = jnp.dot(x_ref[...], wt_ref[...], preferred_element_type=jnp.float32)
    ang = pos_ref[...] * freq_ref[...]
    rot = jnp.where(sel_ref[...] != 0.0, jnp.sin(ang), jnp.cos(ang))
    o_ref[...] = y * rot


def _proj_rope(x, wt, posf):
    R, D = x.shape
    N = wt.shape[1]
    blk = 512
    return pl.pallas_call(
        _proj_rope_body,
        grid=(R // blk,),
        in_specs=[
            pl.BlockSpec((blk, D), lambda i: (i, 0)),
            pl.BlockSpec((D, N), lambda i: (0, 0)),
            pl.BlockSpec((blk, 1), lambda i: (i, 0)),
            pl.BlockSpec((1, N), lambda i: (0, 0)),
            pl.BlockSpec((1, N), lambda i: (0, 0)),
        ],
        out_specs=pl.BlockSpec((blk, N), lambda i: (i, 0)),
        out_shape=jax.ShapeDtypeStruct((R, N), jnp.float32),
    )(x, wt, posf, jnp.asarray(_FREQ_ROW), jnp.asarray(_SIN_SEL))


_QB = 512
_KC = 512


def _attn_body(q_ref, k_ref, v_ref, o_ref, m_ref, l_ref, acc_ref, *, nv):
    j = pl.program_id(2)

    @pl.when(j == 0)
    def _init():
        m_ref[...] = jnp.full((_QB, H), -jnp.inf, dtype=jnp.float32)
        l_ref[...] = jnp.zeros((_QB, H), dtype=jnp.float32)
        acc_ref[...] = jnp.zeros((_QB, H * DH), dtype=jnp.float32)

    scale = np.float32(1.0 / np.sqrt(DH))
    for h in range(H):
        sl = pl.ds(h * DH, DH)
        qh = q_ref[:, sl] * scale
        kh = k_ref[:, sl]
        s = lax.dot_general(
            qh, kh, (((1,), (1,)), ((), ())), preferred_element_type=jnp.float32
        )
        m_old = m_ref[:, pl.ds(h, 1)]
        m_new = jnp.maximum(m_old, jnp.max(s, axis=1, keepdims=True))
        p = jnp.exp(s - m_new)
        alpha = jnp.exp(m_old - m_new)
        l_ref[:, pl.ds(h, 1)] = l_ref[:, pl.ds(h, 1)] * alpha + jnp.sum(
            p, axis=1, keepdims=True
        )
        acc_ref[:, sl] = acc_ref[:, sl] * alpha + jnp.dot(
            p, v_ref[:, sl], preferred_element_type=jnp.float32
        )
        m_ref[:, pl.ds(h, 1)] = m_new

    @pl.when(j == nv - 1)
    def _fin():
        for h in range(H):
            sl = pl.ds(h * DH, DH)
            o_ref[:, sl] = acc_ref[:, sl] * (1.0 / l_ref[:, pl.ds(h, 1)])


def _attn(q2, k2, v2, B, V):
    nq = CAP // _QB
    nv = V // _KC
    body = functools.partial(_attn_body, nv=nv)
    return pl.pallas_call(
        body,
        grid=(B, nq, nv),
        in_specs=[
            pl.BlockSpec((_QB, H * DH), lambda b, i, j: (b * nq + i, 0)),
            pl.BlockSpec((_KC, H * DH), lambda b, i, j: (b * nv + j, 0)),
            pl.BlockSpec((_KC, H * DH), lambda b, i, j: (b * nv + j, 0)),
        ],
        out_specs=pl.BlockSpec((_QB, H * DH), lambda b, i, j: (b * nq + i, 0)),
        out_shape=jax.ShapeDtypeStruct((B * CAP, H * DH), jnp.float32),
        scratch_shapes=[
            pltpu.VMEM((_QB, H), jnp.float32),
            pltpu.VMEM((_QB, H), jnp.float32),
            pltpu.VMEM((_QB, H * DH), jnp.float32),
        ],
    )(q2, k2, v2)


def _outproj_body(x_ref, wt_ref, tv_ref, o_ref):
    y = jnp.dot(x_ref[...], wt_ref[...], preferred_element_type=jnp.float32)
    o_ref[...] = y * tv_ref[...]


def _outproj(x, wt, tv):
    R, N = x.shape
    D = wt.shape[1]
    blk = 512
    return pl.pallas_call(
        _outproj_body,
        grid=(R // blk,),
        in_specs=[
            pl.BlockSpec((blk, N), lambda i: (i, 0)),
            pl.BlockSpec((N, D), lambda i: (0, 0)),
            pl.BlockSpec((blk, 1), lambda i: (i, 0)),
        ],
        out_specs=pl.BlockSpec((blk, D), lambda i: (i, 0)),
        out_shape=jax.ShapeDtypeStruct((R, D), jnp.float32),
    )(x, wt, tv)


_NC, _NS = 2, 16
_NW = _NC * _NS


def _sc_gather(table, gidx):
    Rt, D = table.shape
    N = gidx.shape[0]
    per = N // _NW
    mesh = plsc.VectorSubcoreMesh(core_axis_name="c", subcore_axis_name="s")

    @functools.partial(
        pl.kernel,
        mesh=mesh,
        out_type=jax.ShapeDtypeStruct((N, D), jnp.float32),
        scratch_types=[
            pltpu.VMEM((per,), jnp.int32),
            pltpu.VMEM((per, D), jnp.float32),
            pltpu.SemaphoreType.DMA,
        ],
    )
    def k(table_hbm, idx_hbm, out_hbm, idx_v, rows_v, sem):
        wid = lax.axis_index("s") * _NC + lax.axis_index("c")
        base = wid * per
        pltpu.sync_copy(idx_hbm.at[pl.ds(base, per)], idx_v)
        pltpu.async_copy(table_hbm.at[idx_v], rows_v, sem).wait()
        pltpu.sync_copy(rows_v, out_hbm.at[pl.ds(base, per)])

    return k(table, gidx)


def _sc_scatter(src, gidx, R, D):
    N = gidx.shape[0]
    zper = (R // _NC) // _NS
    per = N // _NW
    zrows = jnp.zeros((per, D), jnp.float32)
    mesh = plsc.VectorSubcoreMesh(core_axis_name="c", subcore_axis_name="s")

    @functools.partial(
        pl.kernel,
        mesh=mesh,
        out_type=jax.ShapeDtypeStruct((R, D), jnp.float32),
        scratch_types=[
            pltpu.VMEM((per,), jnp.int32),
            pltpu.VMEM((per, D), jnp.float32),
            pltpu.SemaphoreType.DMA,
        ],
    )
    def k(src_hbm, idx_hbm, zeros_hbm, out_hbm, idx_v, rows_v, sem):
        c = lax.axis_index("c")
        s = lax.axis_index("s")
        zbase = c * (R // _NC) + s * zper
        pltpu.sync_copy(zeros_hbm, rows_v)
        for j in range(zper // per):
            pltpu.sync_copy(rows_v, out_hbm.at[pl.ds(zbase + j * per, per)])
        plsc.subcore_barrier()
        gbase = c * (N // _NC) + s * per
        pltpu.sync_copy(idx_hbm.at[pl.ds(gbase, per)], idx_v)
        pltpu.sync_copy(src_hbm.at[pl.ds(gbase, per)], rows_v)
        pltpu.async_copy(rows_v, out_hbm.at[idx_v], sem).wait()

    return k(src, gidx, zrows)


def kernel(query_seq, value_seq, W_router, W_q, W_kv, W_out):
    B, Q, D = query_seq.shape
    V = value_seq.shape[1]
    q2 = query_seq.reshape(B * Q, D)
    v2 = value_seq.reshape(B * V, D)

    rw = _router(q2, W_router.T)
    top_vals, top_idx = lax.top_k(rw.reshape(B, Q), CAP)
    gidx = (top_idx + jnp.arange(B, dtype=top_idx.dtype)[:, None] * Q).reshape(-1)
    gidx = gidx.astype(jnp.int32)

    resampled = _sc_gather(q2, gidx)

    kposf = jnp.mod(jnp.arange(B * V), V).astype(jnp.float32).reshape(-1, 1)
    qposf = top_idx.reshape(-1, 1).astype(jnp.float32)

    kp = _proj_rope(v2, W_kv[: H * DH].T, kposf)
    vp = _proj(v2, W_kv[H * DH :].T)
    qp = _proj_rope(resampled, W_q.T, qposf)

    att = _attn(qp, kp, vp, B, V)
    src = _outproj(att, W_out.T, top_vals.reshape(-1, 1))

    out2 = _sc_scatter(src, gidx, B * Q, D)
    return out2.reshape(B, Q, D)
